# baseline (speedup 1.0000x reference)
import functools

import jax
import jax.numpy as jnp
import numpy as np
from jax.experimental import pallas as pl
from jax.experimental.pallas import tpu as pltpu

B = 20
L = 500
N = B * L
NC = 4
HALF = 64
HID = 128
KNN = 9
ATOM = 16
NL = 3
D3 = NC * 3
S2 = 2


def _egnn_kernel(e2_ref, fm_ref, emod_ref, x_ref, cc_ref, cr_ref, pct_ref,
                 ch_ref, gw_ref, ex_ref, wfm_ref, wcat_ref, w1c_ref,
                 w1ch_ref, we2_ref, wh1b_ref, wh2_ref, wx1_ref, wx2_ref,
                 wp1_ref, bp1_ref, wp2_ref, bp2_ref, out_ref):
    f32 = jnp.float32
    R = range(S2)

    def mm(a, w):
        return jnp.dot(a, w, preferred_element_type=f32)

    rows = jax.lax.broadcasted_iota(jnp.int32, (L, L), 0)
    lanes_i = jax.lax.broadcasted_iota(jnp.int32, (L, L), 1)
    diag = jnp.where(rows == lanes_i, f32(1e9), f32(0.0))
    hs, xs, d2s = [], [], []
    for s in R:
        hhalf = e2_ref[s] + mm(fm_ref[s], wfm_ref[...])
        hs.append(jnp.concatenate([hhalf, emod_ref[s]], axis=1))
        xs.append(x_ref[s])
        d2 = diag
        cc = cc_ref[s]
        cr = cr_ref[s]
        for d in range(3):
            diff = cc[:, d:d + 1] - cr[d:d + 1, :]
            d2 = d2 + diff * diff
        d2s.append(d2)

    lane_f = lanes_i.astype(f32)
    ohs = [[] for _ in R]
    for _ in range(KNN):
        for s in R:
            mn = jnp.min(d2s[s], axis=1, keepdims=True)
            am = jnp.min(jnp.where(d2s[s] <= mn, lane_f, f32(2 * L)),
                         axis=1, keepdims=True)
            pk = (lane_f == am).astype(f32)
            ohs[s].append(pk)
            d2s[s] = d2s[s] + pk * f32(1e9)

    gw = gw_ref[...]
    ex = ex_ref[...]
    ch = ch_ref[...]

    for l in range(NL):
        w2 = we2_ref[l]
        wh2 = wh2_ref[l]
        wx1 = wx1_ref[l]
        ch_bias = mm(ch, w1ch_ref[l])
        wx2ex = mm(wx2_ref[l], ex)
        h_bases, h_selfs, gtabs = [], [], []
        aggs, xaccs = [], []
        for s in R:
            hp = mm(hs[s], wcat_ref[l])
            h_bases.append(hp[:, 0:HID] + ch_bias)
            h_selfs.append(hp[:, 2 * HID:3 * HID])
            gtabs.append(jnp.concatenate([hp[:, HID:2 * HID], xs[s]],
                                         axis=1))
            aggs.append(jnp.zeros((L, HID), dtype=f32))
            xaccs.append(jnp.zeros((L, D3), dtype=f32))
        for k in range(KNN):
            for s in R:
                g = mm(ohs[s][k], gtabs[s])
                hcw = g[:, 0:HID]
                xc = g[:, HID:HID + D3]
                diff = xs[s] - xc
                radial = mm(diff * diff, gw)
                pre = h_bases[s] + hcw + mm(radial, w1c_ref[l])
                m = jax.nn.silu(mm(jax.nn.silu(pre), w2))
                aggs[s] = aggs[s] + m
                xaccs[s] = xaccs[s] + diff * mm(
                    jax.nn.silu(mm(m, wx1)), wx2ex)
        for s in R:
            hs[s] = hs[s] + mm(jax.nn.silu(h_selfs[s]
                                           + mm(aggs[s], wh1b_ref[l])), wh2)
            xs[s] = xs[s] + xaccs[s] / f32(KNN)

    sel = jax.lax.broadcasted_iota(jnp.int32, (L, 1), 0) % 10 == 0
    for s in R:
        t = jax.nn.silu(hs[s])
        t = jax.nn.silu(mm(t, wp1_ref[...]) + bp1_ref[...])
        logits = mm(t, wp2_ref[...]) + bp2_ref[...]
        probs = jax.nn.sigmoid(logits)
        d = probs - pct_ref[s]
        ad = jnp.abs(d)
        term = jnp.where(ad < f32(1.0), f32(0.5) * d * d, ad - f32(0.5))
        loss_s = jnp.sum(jnp.where(sel, term, f32(0.0)))
        out_ref[s] = loss_s[None, None]


@functools.partial(jax.jit, static_argnames=())
def kernel(S, X, rna_pos, sec_pos, lengths, pct, marker, smask, atom_mask,
           mod_mask, rna_raw, chain_id, mod, cc, FM, W_seq, W_pos, W_mod,
           W_fm, atom_emb, atom_w, We1, We2, Wh1, Wh2, Wx1, Wx2, Wp1, bp1,
           Wp2, bp2):
    f32 = jnp.float32
    aw = jax.nn.softmax(atom_w)

    e2 = (W_seq[S] + W_pos[rna_pos]).reshape(B, L, HALF)
    emod = W_mod[mod].reshape(B, L, HALF)
    fm = FM.reshape(B, L, -1)

    cent = jnp.einsum('c,ncd->nd', aw, X)
    centc = cent.reshape(B, L, 3)
    centr = jnp.transpose(centc, (0, 2, 1))

    x12 = X.reshape(B, L, D3)

    pctf = jnp.zeros((N,), f32).at[::10].set(pct).reshape(B, L, 1)

    gmask = np.zeros((D3, NC), np.float32)
    for i in range(D3):
        gmask[i, i // 3] = 1.0
    gw = jnp.asarray(gmask) * aw[None, :]
    ex = jnp.asarray(gmask.T)
    ch = (aw @ atom_emb).reshape(1, ATOM)

    bp1r = bp1.reshape(1, HID)
    bp2r = bp2.reshape(1, 1)

    wcat = jnp.concatenate([We1[:, 0:HID, :], We1[:, HID:2 * HID, :],
                            Wh1[:, 0:HID, :]], axis=-1)
    w1c = We1[:, 2 * HID:2 * HID + NC, :]
    w1ch = We1[:, 2 * HID + NC:, :]
    wh1b = Wh1[:, HID:, :]

    def full_spec(arr):
        nd = arr.ndim
        return pl.BlockSpec(arr.shape, lambda b: (0,) * nd)

    def seq_spec(arr):
        return pl.BlockSpec((S2,) + arr.shape[1:],
                            lambda b: (b,) + (0,) * (arr.ndim - 1))

    per_seq = [e2, fm, emod, x12, centc, centr, pctf]
    shared = [ch, gw, ex, W_fm, wcat, w1c, w1ch, We2, wh1b, Wh2, Wx1, Wx2,
              Wp1, bp1r, Wp2, bp2r]

    out = pl.pallas_call(
        _egnn_kernel,
        grid=(B // S2,),
        in_specs=[seq_spec(a) for a in per_seq] + [full_spec(a)
                                                   for a in shared],
        out_specs=pl.BlockSpec((S2, 1, 1), lambda b: (b, 0, 0)),
        out_shape=jax.ShapeDtypeStruct((B, 1, 1), f32),
        compiler_params=pltpu.CompilerParams(
            dimension_semantics=("parallel",)),
    )(*per_seq, *shared)
    return jnp.sum(out) / f32(N // 10)


# speedup vs baseline: 1.2070x; 1.2070x over previous
import functools

import jax
import jax.numpy as jnp
import numpy as np
from jax.experimental import pallas as pl
from jax.experimental.pallas import tpu as pltpu

B = 20
L = 500
N = B * L
NC = 4
HALF = 64
HID = 128
KNN = 9
ATOM = 16
NL = 3
D3 = NC * 3


def _egnn_kernel(e2_ref, fm_ref, emod_ref, x_ref, cc_ref, cr_ref, pct_ref,
                 ch_ref, gw_ref, ex_ref, wfm_ref, wcat_ref, w1c_ref,
                 w1ch_ref, we2_ref, wh1b_ref, wh2_ref, wx1_ref, wx2_ref,
                 wp1_ref, bp1_ref, wp2_ref, bp2_ref, out_ref):
    b = pl.program_id(0)
    f32 = jnp.float32

    hhalf = e2_ref[0] + jnp.dot(fm_ref[0], wfm_ref[...],
                                preferred_element_type=f32)
    h = jnp.concatenate([hhalf, emod_ref[0]], axis=1)
    x = x_ref[0]

    cc = cc_ref[0]
    cr = cr_ref[0]
    rows = jax.lax.broadcasted_iota(jnp.int32, (L, L), 0)
    lanes_i = jax.lax.broadcasted_iota(jnp.int32, (L, L), 1)
    d2 = jnp.where(rows == lanes_i, f32(1e9), f32(0.0))
    for d in range(3):
        diff = cc[:, d:d + 1] - cr[d:d + 1, :]
        d2 = d2 + diff * diff

    lane_f = lanes_i.astype(f32)
    onehots = []
    for _ in range(KNN):
        mn = jnp.min(d2, axis=1, keepdims=True)
        am = jnp.min(jnp.where(d2 <= mn, lane_f, f32(2 * L)), axis=1,
                     keepdims=True)
        pk = (lane_f == am).astype(f32)
        onehots.append(pk)
        d2 = d2 + pk * f32(1e9)

    gw = gw_ref[...]
    ex = ex_ref[...]
    ch = ch_ref[...]

    def mm(a, w):
        return jnp.dot(a, w, preferred_element_type=f32)

    for l in range(NL):
        w2 = we2_ref[l]
        wh2 = wh2_ref[l]
        wx1 = wx1_ref[l]
        wx2 = wx2_ref[l]
        hp = mm(h, wcat_ref[l])
        h_row_w = hp[:, 0:HID]
        h_col_w = hp[:, HID:2 * HID]
        h_self_w = hp[:, 2 * HID:3 * HID]
        ch_bias = mm(ch, w1ch_ref[l])
        wx2ex = mm(wx2, ex)
        h_base = h_row_w + ch_bias
        gtab = jnp.concatenate([h_col_w, x], axis=1)
        agg = jnp.zeros((L, HID), dtype=f32)
        xacc = jnp.zeros((L, D3), dtype=f32)
        for k in range(KNN):
            g = mm(onehots[k], gtab)
            hcw = g[:, 0:HID]
            xc = g[:, HID:HID + D3]
            diff = x - xc
            radial = mm(diff * diff, gw)
            pre = h_base + hcw + mm(radial, w1c_ref[l])
            m = jax.nn.silu(mm(jax.nn.silu(pre), w2))
            agg = agg + m
            xacc = xacc + diff * mm(jax.nn.silu(mm(m, wx1)), wx2ex)
        h = h + mm(jax.nn.silu(h_self_w + mm(agg, wh1b_ref[l])), wh2)
        x = x + xacc / f32(KNN)

    t = jax.nn.silu(h)
    t = jax.nn.silu(mm(t, wp1_ref[...]) + bp1_ref[...])
    logits = mm(t, wp2_ref[...]) + bp2_ref[...]
    probs = jax.nn.sigmoid(logits)
    d = probs - pct_ref[0]
    ad = jnp.abs(d)
    term = jnp.where(ad < f32(1.0), f32(0.5) * d * d, ad - f32(0.5))
    sel = jax.lax.broadcasted_iota(jnp.int32, (L, 1), 0) % 10 == 0
    loss_b = jnp.sum(jnp.where(sel, term, f32(0.0)))
    out_ref[...] = loss_b[None, None, None]


@functools.partial(jax.jit, static_argnames=())
def kernel(S, X, rna_pos, sec_pos, lengths, pct, marker, smask, atom_mask,
           mod_mask, rna_raw, chain_id, mod, cc, FM, W_seq, W_pos, W_mod,
           W_fm, atom_emb, atom_w, We1, We2, Wh1, Wh2, Wx1, Wx2, Wp1, bp1,
           Wp2, bp2):
    f32 = jnp.float32
    aw = jax.nn.softmax(atom_w)

    e2 = (W_seq[S] + W_pos[rna_pos]).reshape(B, L, HALF)
    emod = W_mod[mod].reshape(B, L, HALF)
    fm = FM.reshape(B, L, -1)

    cent = jnp.einsum('c,ncd->nd', aw, X)
    centc = cent.reshape(B, L, 3)
    centr = jnp.transpose(centc, (0, 2, 1))

    x12 = X.reshape(B, L, D3)

    pctf = jnp.zeros((N,), f32).at[::10].set(pct).reshape(B, L, 1)

    gmask = np.zeros((D3, NC), np.float32)
    for i in range(D3):
        gmask[i, i // 3] = 1.0
    gw = jnp.asarray(gmask) * aw[None, :]
    ex = jnp.asarray(gmask.T)
    ch = (aw @ atom_emb).reshape(1, ATOM)

    bp1r = bp1.reshape(1, HID)
    bp2r = bp2.reshape(1, 1)

    wcat = jnp.concatenate([We1[:, 0:HID, :], We1[:, HID:2 * HID, :],
                            Wh1[:, 0:HID, :]], axis=-1)
    w1c = We1[:, 2 * HID:2 * HID + NC, :]
    w1ch = We1[:, 2 * HID + NC:, :]
    wh1b = Wh1[:, HID:, :]

    def full_spec(arr):
        nd = arr.ndim
        return pl.BlockSpec(arr.shape, lambda b: (0,) * nd)

    def seq_spec(arr):
        return pl.BlockSpec((1,) + arr.shape[1:],
                            lambda b: (b,) + (0,) * (arr.ndim - 1))

    per_seq = [e2, fm, emod, x12, centc, centr, pctf]
    shared = [ch, gw, ex, W_fm, wcat, w1c, w1ch, We2, wh1b, Wh2, Wx1, Wx2,
              Wp1, bp1r, Wp2, bp2r]

    out = pl.pallas_call(
        _egnn_kernel,
        grid=(B,),
        in_specs=[seq_spec(a) for a in per_seq] + [full_spec(a)
                                                   for a in shared],
        out_specs=pl.BlockSpec((1, 1, 1), lambda b: (b, 0, 0)),
        out_shape=jax.ShapeDtypeStruct((B, 1, 1), f32),
        compiler_params=pltpu.CompilerParams(
            dimension_semantics=("parallel",)),
    )(*per_seq, *shared)
    return jnp.sum(out) / f32(N // 10)
